# baseline (speedup 1.0000x reference)
import functools

import jax
import jax.numpy as jnp
from jax import lax
from jax.experimental import pallas as pl
from jax.experimental.pallas import tpu as pltpu
from jax.experimental.pallas import tpu_sc as plsc

_NC = 2
_NS = 16
_L = 16
_R = 16


def _sc_body(x_hbm, pos_hbm, out_hbm, pbufs, obufs, spos, sins, souts):
    B, T, D = x_hbm.shape
    tw = T // (_NC * _NS)
    nb = tw // _R
    wid = lax.axis_index("s") * _NC + lax.axis_index("c")
    t0 = wid * tw

    def do_block(tb, pbuf, pos_sem, pos_next):
        hin = [None, None, None]
        hout = [None, None, None]
        hin[0] = pltpu.async_copy(x_hbm.at[0, pl.ds(tb, _R)], obufs[0],
                                  sins[0])
        nsrc, ndst, nsem = pos_next
        pltpu.async_copy(nsrc, ndst, nsem)
        pltpu.make_async_copy(pos_hbm.at[pl.ds(tb, _R)], pbuf, pos_sem).wait()
        for b in range(B):
            cur = b % 3
            if b + 1 < B:
                nxt = (b + 1) % 3
                if hout[nxt] is not None:
                    hout[nxt].wait()
                    hout[nxt] = None
                hin[nxt] = pltpu.async_copy(
                    x_hbm.at[b + 1, pl.ds(tb, _R)], obufs[nxt], sins[nxt])
            hin[cur].wait()

            ob = obufs[cur]

            def row_body(r, ob=ob, pbuf=pbuf):
                for jc in range(D // (_L * 16)):
                    for u in range(16):
                        off = jc * (_L * 16) + u * _L
                        pv = pbuf[r, pl.ds(off, _L)]
                        plsc.addupdate(ob.at[r, pl.ds(off, _L)], pv)

            plsc.parallel_loop(0, _R, 1, unroll=2)(row_body)
            hout[cur] = pltpu.async_copy(
                ob, out_hbm.at[b, pl.ds(tb, _R)], souts[cur])
        for k in (1, 2, 0):
            if hout[k] is not None:
                hout[k].wait()

    pltpu.async_copy(pos_hbm.at[pl.ds(t0, _R)], pbufs[0], spos[0])

    def pair_loop(j, carry):
        tb0 = t0 + (2 * j) * _R
        tb1 = tb0 + _R
        tb2 = jnp.minimum(t0 + (2 * j + 2) * _R, T - _R)
        do_block(tb0, pbufs[0], spos[0],
                 (pos_hbm.at[pl.ds(tb1, _R)], pbufs[1], spos[1]))
        do_block(tb1, pbufs[1], spos[1],
                 (pos_hbm.at[pl.ds(tb2, _R)], pbufs[0], spos[0]))
        return carry

    lax.fori_loop(0, nb // 2, pair_loop, 0)
    pltpu.make_async_copy(
        pos_hbm.at[pl.ds(t0, _R)], pbufs[0], spos[0]).wait()


def kernel(x, pos_embed):
    B, T, D = x.shape
    mesh = plsc.VectorSubcoreMesh(core_axis_name="c", subcore_axis_name="s")
    k = pl.kernel(
        _sc_body,
        out_type=jax.ShapeDtypeStruct((B, T, D), x.dtype),
        mesh=mesh,
        scratch_types=[
            [pltpu.VMEM((_R, D), jnp.float32) for _ in range(2)],
            [pltpu.VMEM((_R, D), jnp.float32) for _ in range(3)],
            [pltpu.SemaphoreType.DMA for _ in range(2)],
            [pltpu.SemaphoreType.DMA for _ in range(3)],
            [pltpu.SemaphoreType.DMA for _ in range(3)],
        ],
    )
    return k(x, pos_embed[:T])


# speedup vs baseline: 1.1890x; 1.1890x over previous
import functools

import jax
import jax.numpy as jnp
from jax import lax
from jax.experimental import pallas as pl
from jax.experimental.pallas import tpu as pltpu
from jax.experimental.pallas import tpu_sc as plsc

_NC = 2
_NS = 16
_L = 16
_R = 16
_NBUF = 4


def _sc_body(x_hbm, pos_hbm, out_hbm, pbufs, obufs, spos, sins, souts):
    B, T, D = x_hbm.shape
    tw = T // (_NC * _NS)
    nb = tw // _R
    wid = lax.axis_index("s") * _NC + lax.axis_index("c")
    t0 = wid * tw

    def add_pos(ob, pbuf):
        def row_body(r):
            for jc in range(D // (_L * 16)):
                for u in range(16):
                    off = jc * (_L * 16) + u * _L
                    pv = pbuf[r, pl.ds(off, _L)]
                    plsc.addupdate(ob.at[r, pl.ds(off, _L)], pv)

        plsc.parallel_loop(0, _R, 1, unroll=1)(row_body)

    def wait_out(k):
        pltpu.make_async_copy(
            obufs[k], out_hbm.at[0, pl.ds(t0, _R)], souts[k]).wait()

    def do_block(tb, pbuf, pos_sem, pos_next, ready):
        hin = [None] * B
        pl.when(ready)(lambda: wait_out(0))
        hin[0] = pltpu.async_copy(x_hbm.at[0, pl.ds(tb, _R)], obufs[0],
                                  sins[0])
        nsrc, ndst, nsem = pos_next
        pltpu.async_copy(nsrc, ndst, nsem)
        pltpu.make_async_copy(pos_hbm.at[pl.ds(tb, _R)], pbuf, pos_sem).wait()
        for b in range(B):
            if b + 1 < B:
                pl.when(ready)(lambda b=b: wait_out(b + 1))
                hin[b + 1] = pltpu.async_copy(
                    x_hbm.at[b + 1, pl.ds(tb, _R)], obufs[b + 1], sins[b + 1])
            hin[b].wait()
            add_pos(obufs[b], pbuf)
            pltpu.async_copy(obufs[b], out_hbm.at[b, pl.ds(tb, _R)], souts[b])

    pltpu.async_copy(pos_hbm.at[pl.ds(t0, _R)], pbufs[0], spos[0])

    def pair_loop(j, carry):
        tb0 = t0 + (2 * j) * _R
        tb1 = tb0 + _R
        tb2 = jnp.minimum(t0 + (2 * j + 2) * _R, T - _R)
        do_block(tb0, pbufs[0], spos[0],
                 (pos_hbm.at[pl.ds(tb1, _R)], pbufs[1], spos[1]), j > 0)
        do_block(tb1, pbufs[1], spos[1],
                 (pos_hbm.at[pl.ds(tb2, _R)], pbufs[0], spos[0]), j >= 0)
        return carry

    lax.fori_loop(0, nb // 2, pair_loop, 0)
    for k in range(_NBUF):
        wait_out(k)
    pltpu.make_async_copy(
        pos_hbm.at[pl.ds(t0, _R)], pbufs[0], spos[0]).wait()


def kernel(x, pos_embed):
    B, T, D = x.shape
    mesh = plsc.VectorSubcoreMesh(core_axis_name="c", subcore_axis_name="s")
    k = pl.kernel(
        _sc_body,
        out_type=jax.ShapeDtypeStruct((B, T, D), x.dtype),
        mesh=mesh,
        scratch_types=[
            [pltpu.VMEM((_R, D), jnp.float32) for _ in range(2)],
            [pltpu.VMEM((_R, D), jnp.float32) for _ in range(_NBUF)],
            [pltpu.SemaphoreType.DMA for _ in range(2)],
            [pltpu.SemaphoreType.DMA for _ in range(_NBUF)],
            [pltpu.SemaphoreType.DMA for _ in range(_NBUF)],
        ],
    )
    return k(x, pos_embed[:T])


# speedup vs baseline: 2.4299x; 2.0437x over previous
import jax
import jax.numpy as jnp
from jax.experimental import pallas as pl


def _add_body(x_ref, pos_ref, out_ref):
    out_ref[...] = x_ref[...] + pos_ref[...]


def kernel(x, pos_embed):
    B, T, D = x.shape
    BT = 2048
    grid = (T // BT, B)
    return pl.pallas_call(
        _add_body,
        grid=grid,
        in_specs=[
            pl.BlockSpec((1, BT, D), lambda t, b: (b, t, 0)),
            pl.BlockSpec((BT, D), lambda t, b: (t, 0)),
        ],
        out_specs=pl.BlockSpec((1, BT, D), lambda t, b: (b, t, 0)),
        out_shape=jax.ShapeDtypeStruct((B, T, D), x.dtype),
    )(x, pos_embed[:T])
